# baseline (speedup 1.0000x reference)
import functools

import jax
import jax.numpy as jnp
from jax import lax
from jax.experimental import pallas as pl
from jax.experimental.pallas import tpu as pltpu
from jax.experimental.pallas import tpu_sc as plsc

N = 10000
E = 320000
F = 128
FH = F // 2
NPAD = 10112
NT = 32
CH = 64
ET = E + N
NJ0 = 184
NJ1 = 140
NJMAX = NJ0
TOTC = 16 * (NJ0 + NJ1)
TOTCP = TOTC + NJMAX
EP = TOTC * CH
STRIPE = NPAD // 16

_mesh = plsc.VectorSubcoreMesh(core_axis_name="c", subcore_axis_name="s")
f32 = jnp.float32

_params = pltpu.CompilerParams(needs_layout_passes=False,
                               use_tc_tiling_on_sc=False)


def _wid():
    return lax.axis_index("c") * 16 + lax.axis_index("s")


def _tile_chunks():
    cid = lax.axis_index("c")
    sid = lax.axis_index("s")
    roff = jnp.where(cid == 0, sid * NJ0, 16 * NJ0 + sid * NJ1)
    nj = jnp.where(cid == 0, NJ0, NJ1)
    return roff, nj


def _zero_hist(hist):
    zv = jnp.zeros((16,), f32)

    @pl.loop(0, NPAD // 16)
    def _(i):
        hist[pl.ds(i * 16, 16)] = zv


def _zero_stripe(zbuf, acc, base):
    nfull = STRIPE // CH
    for i in range(nfull):
        pltpu.sync_copy(zbuf, acc.at[pl.ds(base + i * CH, CH)])
    rem = STRIPE - nfull * CH
    if rem:
        pltpu.sync_copy(zbuf.at[pl.ds(0, rem)],
                        acc.at[pl.ds(base + nfull * CH, rem)])


@functools.partial(
    pl.kernel,
    compiler_params=_params,
    out_type=jax.ShapeDtypeStruct((NT, NPAD), f32),
    mesh=_mesh,
    scratch_types=[
        pltpu.VMEM((NJMAX, CH), jnp.int32),
        pltpu.VMEM((NPAD,), f32),
    ],
)
def _sc_deg(dst_hbm, out_hbm, dstv, hist):
    roff, nj = _tile_chunks()
    pltpu.sync_copy(dst_hbm.at[pl.ds(roff, NJMAX)], dstv)
    _zero_hist(hist)
    ones = jnp.ones((16,), f32)

    @pl.loop(0, nj)
    def _(j):
        for k in range(CH // 16):
            d16 = dstv[j, pl.ds(k * 16, 16)]
            plsc.addupdate_scatter(hist, [d16], ones)

    pltpu.sync_copy(hist, out_hbm.at[_wid()])


@functools.partial(
    pl.kernel,
    compiler_params=_params,
    out_type=jax.ShapeDtypeStruct((2, 2, NPAD, FH), f32),
    mesh=_mesh,
    scratch_types=[
        pltpu.VMEM((NJMAX, CH), jnp.int32),
        pltpu.VMEM((NJMAX, CH), jnp.int32),
        pltpu.VMEM((CH, FH), f32),
        pltpu.VMEM((CH, FH), f32),
        pltpu.VMEM_SHARED((NPAD, FH), f32),
        pltpu.SemaphoreType.DMA,
        pltpu.SemaphoreType.DMA,
    ],
)
def _sc_gcn(src2_hbm, dst_hbm, g2_hbm, zf_hbm, out_hbm, srcv, dstv,
            bufa, bufb, acc, sema, semb):
    cid = lax.axis_index("c")
    sid = lax.axis_index("s")
    base = sid * STRIPE
    roff, nj = _tile_chunks()
    nj2 = nj // 2
    pltpu.sync_copy(dst_hbm.at[pl.ds(roff, NJMAX)], dstv)

    for p in range(2):
        pltpu.sync_copy(src2_hbm.at[p, pl.ds(roff, NJMAX)], srcv)
        pltpu.sync_copy(zf_hbm, bufa)
        _zero_stripe(bufa, acc, base)
        plsc.subcore_barrier()

        pltpu.async_copy(g2_hbm.at[srcv.at[0]], bufa, sema)
        pltpu.async_copy(g2_hbm.at[srcv.at[1]], bufb, semb)

        @pl.loop(0, nj2)
        def _(jj):
            j = 2 * jj
            pltpu.make_async_copy(g2_hbm.at[srcv.at[j]], bufa, sema).wait()
            pltpu.sync_copy(bufa, acc.at[dstv.at[j]], add=True)

            @pl.when(jj < nj2 - 1)
            def _():
                pltpu.async_copy(g2_hbm.at[srcv.at[j + 2]], bufa, sema)

            pltpu.make_async_copy(g2_hbm.at[srcv.at[j + 1]], bufb,
                                  semb).wait()
            pltpu.sync_copy(bufb, acc.at[dstv.at[j + 1]], add=True)

            @pl.when(jj < nj2 - 1)
            def _():
                pltpu.async_copy(g2_hbm.at[srcv.at[j + 3]], bufb, semb)

        plsc.subcore_barrier()
        pltpu.sync_copy(acc.at[pl.ds(base, STRIPE)],
                        out_hbm.at[cid, p, pl.ds(base, STRIPE)])
        if p == 0:
            plsc.subcore_barrier()


def _lane_bcast(v, e):
    idx = jnp.full((16, 1), e, jnp.int32)
    dn = lax.GatherDimensionNumbers(
        offset_dims=(), collapsed_slice_dims=(0,), start_index_map=(0,))
    return lax.gather(v, idx, dn, (1,),
                      mode=lax.GatherScatterMode.PROMISE_IN_BOUNDS)


def _table_max(ref):
    def body(i, m):
        return jnp.maximum(m, ref[pl.ds(i * 16, 16)])
    m = lax.fori_loop(0, NPAD // 16, body,
                      jnp.full((16,), -jnp.inf, f32))
    return jnp.max(m)


@functools.partial(
    pl.kernel,
    compiler_params=_params,
    out_type=(jax.ShapeDtypeStruct((2, 2, NPAD, FH), f32),
              jax.ShapeDtypeStruct((NT, NPAD), f32)),
    mesh=_mesh,
    scratch_types=[
        pltpu.VMEM((NJMAX, CH), jnp.int32),
        pltpu.VMEM((NJMAX, CH), jnp.int32),
        pltpu.VMEM((NJMAX, CH), f32),
        pltpu.VMEM((NPAD,), f32),
        pltpu.VMEM((NPAD,), f32),
        pltpu.VMEM((NPAD,), f32),
        pltpu.VMEM((CH, FH), f32),
        pltpu.VMEM((CH, FH), f32),
        pltpu.VMEM_SHARED((NPAD, FH), f32),
        pltpu.SemaphoreType.DMA,
        pltpu.SemaphoreType.DMA,
    ],
)
def _sc_gat(src2_hbm, dst_hbm, h22_hbm, as_hbm, ad_hbm, zf_hbm,
            out_hbm, den_hbm,
            srcv, dstv, wv, asv, adv, hist, bufa, bufb, acc, sema, semb):
    cid = lax.axis_index("c")
    sid = lax.axis_index("s")
    base = sid * STRIPE
    roff, nj = _tile_chunks()
    nj2 = nj // 2
    pltpu.sync_copy(dst_hbm.at[pl.ds(roff, NJMAX)], dstv)
    pltpu.sync_copy(as_hbm, asv)
    pltpu.sync_copy(ad_hbm, adv)
    _zero_hist(hist)

    cvec = jnp.full((16,), _table_max(asv) + _table_max(adv), f32)

    def process(buf, j, p):
        for k in range(CH // 16):
            if p == 0:
                s16 = lax.shift_right_logical(srcv[j, pl.ds(k * 16, 16)], 1)
                d16 = dstv[j, pl.ds(k * 16, 16)]
                av = plsc.load_gather(asv, [s16])
                bv = plsc.load_gather(adv, [d16])
                z = av + bv
                z = jnp.where(z > 0.0, z, z * jnp.float32(0.2))
                w16 = jnp.exp(z - cvec)
                wv[j, pl.ds(k * 16, 16)] = w16
                plsc.addupdate_scatter(hist, [d16], w16)
            else:
                w16 = wv[j, pl.ds(k * 16, 16)]
            for e in range(16):
                row = k * 16 + e
                br = _lane_bcast(w16, e)
                for r in range(FH // 16):
                    sl = pl.ds(r * 16, 16)
                    buf[row, sl] = buf[row, sl] * br

    for p in range(2):
        pltpu.sync_copy(src2_hbm.at[p, pl.ds(roff, NJMAX)], srcv)
        pltpu.sync_copy(zf_hbm, bufa)
        _zero_stripe(bufa, acc, base)
        plsc.subcore_barrier()

        pltpu.async_copy(h22_hbm.at[srcv.at[0]], bufa, sema)
        pltpu.async_copy(h22_hbm.at[srcv.at[1]], bufb, semb)

        @pl.loop(0, nj2)
        def _(jj):
            j = 2 * jj
            pltpu.make_async_copy(h22_hbm.at[srcv.at[j]], bufa, sema).wait()
            process(bufa, j, p)
            pltpu.sync_copy(bufa, acc.at[dstv.at[j]], add=True)

            @pl.when(jj < nj2 - 1)
            def _():
                pltpu.async_copy(h22_hbm.at[srcv.at[j + 2]], bufa, sema)

            pltpu.make_async_copy(h22_hbm.at[srcv.at[j + 1]], bufb,
                                  semb).wait()
            process(bufb, j + 1, p)
            pltpu.sync_copy(bufb, acc.at[dstv.at[j + 1]], add=True)

            @pl.when(jj < nj2 - 1)
            def _():
                pltpu.async_copy(h22_hbm.at[srcv.at[j + 3]], bufb, semb)

        plsc.subcore_barrier()
        pltpu.sync_copy(acc.at[pl.ds(base, STRIPE)],
                        out_hbm.at[cid, p, pl.ds(base, STRIPE)])
        if p == 0:
            plsc.subcore_barrier()

    pltpu.sync_copy(hist, den_hbm.at[_wid()])


def _tc1a_body(x_ref, w1_ref, h_ref):
    h_ref[...] = jnp.dot(x_ref[...], w1_ref[...], preferred_element_type=f32)


def _tc1_body(h_ref, degp_ref, g_ref, dinv_ref):
    deg = jnp.sum(degp_ref[...][:, :N], axis=0)[:, None]
    dinv = lax.rsqrt(deg)
    g_ref[...] = h_ref[...] * dinv
    dinv_ref[...] = dinv


def _tc2_body(accp_ref, dinv_ref, b1_ref, w2_ref, att_ref, h2_ref, as_ref,
              ad_ref):
    accp = accp_ref[...]
    acc = jnp.concatenate(
        [accp[0, 0, :N, :] + accp[1, 0, :N, :],
         accp[0, 1, :N, :] + accp[1, 1, :N, :]], axis=1)
    h1 = jnp.maximum(acc * dinv_ref[...] + b1_ref[...], 0.0)
    h2 = jnp.dot(h1, w2_ref[...], preferred_element_type=f32)
    h2_ref[...] = h2
    a2 = jnp.dot(h2, att_ref[...], preferred_element_type=f32)
    pad = jnp.zeros((NPAD - N, 1), f32)
    as_ref[...] = jnp.concatenate([a2[:, 0:1], pad], axis=0)
    ad_ref[...] = jnp.concatenate([a2[:, 1:2], pad], axis=0)


def _tc3_body(nump_ref, denp_ref, b2_ref, out_ref):
    nump = nump_ref[...]
    num = jnp.concatenate(
        [nump[0, 0, :N, :] + nump[1, 0, :N, :],
         nump[0, 1, :N, :] + nump[1, 1, :N, :]], axis=1)
    den = jnp.sum(denp_ref[...][:, :N], axis=0)[:, None]
    out_ref[...] = num / den + b2_ref[...]


_tc1a = pl.pallas_call(
    _tc1a_body,
    out_shape=jax.ShapeDtypeStruct((N, F), f32))
_tc1 = pl.pallas_call(
    _tc1_body,
    out_shape=(jax.ShapeDtypeStruct((N, F), f32),
               jax.ShapeDtypeStruct((N, 1), f32)))
_tc2 = pl.pallas_call(
    _tc2_body,
    out_shape=(jax.ShapeDtypeStruct((N, F), f32),
               jax.ShapeDtypeStruct((NPAD, 1), f32),
               jax.ShapeDtypeStruct((NPAD, 1), f32)))
_tc3 = pl.pallas_call(
    _tc3_body,
    out_shape=jax.ShapeDtypeStruct((N, F), f32))


def kernel(x, edge_index, W1, b1, W2, att_src, att_dst, b2):
    loop = jnp.arange(N, dtype=jnp.int32)
    src = jnp.concatenate([edge_index[0], loop])
    dst = jnp.concatenate([edge_index[1], loop])
    padn = TOTCP * CH - ET
    srcp = jnp.concatenate(
        [src, jnp.zeros((padn,), jnp.int32)]).reshape(TOTCP, CH)
    dstp = jnp.concatenate(
        [dst, jnp.full((padn,), N, jnp.int32)]).reshape(TOTCP, CH)
    src2 = jnp.stack([2 * srcp, 2 * srcp + 1])

    zf = jnp.zeros((CH, FH), f32)
    attmat = jnp.stack([att_src, att_dst], axis=1)

    degp = _sc_deg(dstp)
    h = _tc1a(x, W1)
    g, dinv = _tc1(h, degp)
    accp = _sc_gcn(src2, dstp, g.reshape(2 * N, FH), zf)
    h2, asp, adp = _tc2(accp, dinv, b1.reshape(1, F), W2, attmat)
    nump, denp = _sc_gat(src2, dstp, h2.reshape(2 * N, FH),
                         asp.reshape(NPAD), adp.reshape(NPAD), zf)
    out = _tc3(nump, denp, b2.reshape(1, F))
    return out


# speedup vs baseline: 1.0034x; 1.0034x over previous
import functools

import jax
import jax.numpy as jnp
from jax import lax
from jax.experimental import pallas as pl
from jax.experimental.pallas import tpu as pltpu
from jax.experimental.pallas import tpu_sc as plsc

N = 10000
E = 320000
F = 128
FH = F // 2
NPAD = 10112
NT = 32
CH = 64
ET = E + N
NJ0 = 184
NJ1 = 140
NJMAX = NJ0
TOTC = 16 * (NJ0 + NJ1)
TOTCP = TOTC + NJMAX
EP = TOTC * CH
STRIPE = NPAD // 16

_mesh = plsc.VectorSubcoreMesh(core_axis_name="c", subcore_axis_name="s")
f32 = jnp.float32

_params = pltpu.CompilerParams(needs_layout_passes=False,
                               use_tc_tiling_on_sc=False)


def _wid():
    return lax.axis_index("c") * 16 + lax.axis_index("s")


def _tile_chunks():
    cid = lax.axis_index("c")
    sid = lax.axis_index("s")
    roff = jnp.where(cid == 0, sid * NJ0, 16 * NJ0 + sid * NJ1)
    nj = jnp.where(cid == 0, NJ0, NJ1)
    return roff, nj


def _zero_hist(hist):
    zv = jnp.zeros((16,), f32)

    @pl.loop(0, NPAD // 16)
    def _(i):
        hist[pl.ds(i * 16, 16)] = zv


def _zero_stripe(zbuf, acc, base):
    nfull = STRIPE // CH
    for i in range(nfull):
        pltpu.sync_copy(zbuf, acc.at[pl.ds(base + i * CH, CH)])
    rem = STRIPE - nfull * CH
    if rem:
        pltpu.sync_copy(zbuf.at[pl.ds(0, rem)],
                        acc.at[pl.ds(base + nfull * CH, rem)])


@functools.partial(
    pl.kernel,
    compiler_params=_params,
    out_type=jax.ShapeDtypeStruct((NT, NPAD), f32),
    mesh=_mesh,
    scratch_types=[
        pltpu.VMEM((NJMAX, CH), jnp.int32),
        pltpu.VMEM((NPAD,), f32),
    ],
)
def _sc_deg(dst_hbm, out_hbm, dstv, hist):
    roff, nj = _tile_chunks()
    pltpu.sync_copy(dst_hbm.at[pl.ds(roff, NJMAX)], dstv)
    _zero_hist(hist)
    ones = jnp.ones((16,), f32)

    @pl.loop(0, nj)
    def _(j):
        for k in range(CH // 16):
            d16 = dstv[j, pl.ds(k * 16, 16)]
            plsc.addupdate_scatter(hist, [d16], ones)

    pltpu.sync_copy(hist, out_hbm.at[_wid()])


@functools.partial(
    pl.kernel,
    compiler_params=_params,
    out_type=jax.ShapeDtypeStruct((2, 2, NPAD, FH), f32),
    mesh=_mesh,
    scratch_types=[
        pltpu.VMEM((NJMAX, CH), jnp.int32),
        pltpu.VMEM((NJMAX, CH), jnp.int32),
        pltpu.VMEM((CH, FH), f32),
        pltpu.VMEM((CH, FH), f32),
        pltpu.VMEM_SHARED((NPAD, FH), f32),
        pltpu.SemaphoreType.DMA,
        pltpu.SemaphoreType.DMA,
    ],
)
def _sc_gcn(src2_hbm, dst_hbm, g2_hbm, zf_hbm, out_hbm, srcv, dstv,
            bufa, bufb, acc, sema, semb):
    cid = lax.axis_index("c")
    sid = lax.axis_index("s")
    base = sid * STRIPE
    roff, nj = _tile_chunks()
    nj2 = nj // 2
    pltpu.sync_copy(dst_hbm.at[pl.ds(roff, NJMAX)], dstv)

    for p in range(2):
        pltpu.sync_copy(src2_hbm.at[p, pl.ds(roff, NJMAX)], srcv)
        pltpu.sync_copy(zf_hbm, bufa)
        _zero_stripe(bufa, acc, base)
        plsc.subcore_barrier()

        pltpu.async_copy(g2_hbm.at[srcv.at[0]], bufa, sema)
        pltpu.async_copy(g2_hbm.at[srcv.at[1]], bufb, semb)

        @pl.loop(0, nj2)
        def _(jj):
            j = 2 * jj
            pltpu.make_async_copy(g2_hbm.at[srcv.at[j]], bufa, sema).wait()
            pltpu.sync_copy(bufa, acc.at[dstv.at[j]], add=True)

            @pl.when(jj < nj2 - 1)
            def _():
                pltpu.async_copy(g2_hbm.at[srcv.at[j + 2]], bufa, sema)

            pltpu.make_async_copy(g2_hbm.at[srcv.at[j + 1]], bufb,
                                  semb).wait()
            pltpu.sync_copy(bufb, acc.at[dstv.at[j + 1]], add=True)

            @pl.when(jj < nj2 - 1)
            def _():
                pltpu.async_copy(g2_hbm.at[srcv.at[j + 3]], bufb, semb)

        plsc.subcore_barrier()
        pltpu.sync_copy(acc.at[pl.ds(base, STRIPE)],
                        out_hbm.at[cid, p, pl.ds(base, STRIPE)])
        if p == 0:
            plsc.subcore_barrier()


def _lane_bcast(v, e):
    idx = jnp.full((16, 1), e, jnp.int32)
    dn = lax.GatherDimensionNumbers(
        offset_dims=(), collapsed_slice_dims=(0,), start_index_map=(0,))
    return lax.gather(v, idx, dn, (1,),
                      mode=lax.GatherScatterMode.PROMISE_IN_BOUNDS)


def _table_max(ref):
    def body(i, m):
        return jnp.maximum(m, ref[pl.ds(i * 16, 16)])
    m = lax.fori_loop(0, NPAD // 16, body,
                      jnp.full((16,), -jnp.inf, f32))
    return jnp.max(m)


@functools.partial(
    pl.kernel,
    compiler_params=_params,
    out_type=(jax.ShapeDtypeStruct((2, 2, NPAD, FH), f32),
              jax.ShapeDtypeStruct((NT, NPAD), f32)),
    mesh=_mesh,
    scratch_types=[
        pltpu.VMEM((NJMAX, CH), jnp.int32),
        pltpu.VMEM((NJMAX, CH), jnp.int32),
        pltpu.VMEM((NJMAX, CH), f32),
        pltpu.VMEM((NPAD,), f32),
        pltpu.VMEM((NPAD,), f32),
        pltpu.VMEM((NPAD,), f32),
        pltpu.VMEM((CH, FH), f32),
        pltpu.VMEM((CH, FH), f32),
        pltpu.VMEM_SHARED((NPAD, FH), f32),
        pltpu.SemaphoreType.DMA,
        pltpu.SemaphoreType.DMA,
    ],
)
def _sc_gat(src2_hbm, dst_hbm, h22_hbm, as_hbm, ad_hbm, zf_hbm,
            out_hbm, den_hbm,
            srcv, dstv, wv, asv, adv, hist, bufa, bufb, acc, sema, semb):
    cid = lax.axis_index("c")
    sid = lax.axis_index("s")
    base = sid * STRIPE
    roff, nj = _tile_chunks()
    nj2 = nj // 2
    pltpu.sync_copy(dst_hbm.at[pl.ds(roff, NJMAX)], dstv)
    pltpu.sync_copy(as_hbm, asv)
    pltpu.sync_copy(ad_hbm, adv)
    _zero_hist(hist)

    cvec = jnp.full((16,), _table_max(asv) + _table_max(adv), f32)

    def process(buf, j, p):
        for k in range(CH // 16):
            if p == 0:
                s16 = lax.shift_right_logical(srcv[j, pl.ds(k * 16, 16)], 1)
                d16 = dstv[j, pl.ds(k * 16, 16)]
                av = plsc.load_gather(asv, [s16])
                bv = plsc.load_gather(adv, [d16])
                z = av + bv
                z = jnp.where(z > 0.0, z, z * jnp.float32(0.2))
                w16 = jnp.exp(z - cvec)
                wv[j, pl.ds(k * 16, 16)] = w16
                plsc.addupdate_scatter(hist, [d16], w16)
            else:
                w16 = wv[j, pl.ds(k * 16, 16)]
            for e in range(16):
                row = k * 16 + e
                br = _lane_bcast(w16, e)
                for r in range(FH // 16):
                    sl = pl.ds(r * 16, 16)
                    buf[row, sl] = buf[row, sl] * br

    for p in range(2):
        pltpu.sync_copy(src2_hbm.at[p, pl.ds(roff, NJMAX)], srcv)
        pltpu.sync_copy(zf_hbm, bufa)
        _zero_stripe(bufa, acc, base)
        plsc.subcore_barrier()

        pltpu.async_copy(h22_hbm.at[srcv.at[0]], bufa, sema)
        pltpu.async_copy(h22_hbm.at[srcv.at[1]], bufb, semb)

        @pl.loop(0, nj2)
        def _(jj):
            j = 2 * jj
            pltpu.make_async_copy(h22_hbm.at[srcv.at[j]], bufa, sema).wait()
            process(bufa, j, p)
            pltpu.sync_copy(bufa, acc.at[dstv.at[j]], add=True)

            @pl.when(jj < nj2 - 1)
            def _():
                pltpu.async_copy(h22_hbm.at[srcv.at[j + 2]], bufa, sema)

            pltpu.make_async_copy(h22_hbm.at[srcv.at[j + 1]], bufb,
                                  semb).wait()
            process(bufb, j + 1, p)
            pltpu.sync_copy(bufb, acc.at[dstv.at[j + 1]], add=True)

            @pl.when(jj < nj2 - 1)
            def _():
                pltpu.async_copy(h22_hbm.at[srcv.at[j + 3]], bufb, semb)

        plsc.subcore_barrier()
        pltpu.sync_copy(acc.at[pl.ds(base, STRIPE)],
                        out_hbm.at[cid, p, pl.ds(base, STRIPE)])
        if p == 0:
            plsc.subcore_barrier()

    pltpu.sync_copy(hist, den_hbm.at[_wid()])


def _tc1_body(x_ref, w1_ref, degp_ref, g_ref, dinv_ref):
    deg = jnp.sum(degp_ref[...][:, :N], axis=0)[:, None]
    dinv = lax.rsqrt(deg)
    h = jnp.dot(x_ref[...], w1_ref[...], preferred_element_type=f32)
    g_ref[...] = h * dinv
    dinv_ref[...] = dinv


def _tc2_body(accp_ref, dinv_ref, b1_ref, w2_ref, att_ref, h2_ref, as_ref,
              ad_ref):
    accp = accp_ref[...]
    acc = jnp.concatenate(
        [accp[0, 0, :N, :] + accp[1, 0, :N, :],
         accp[0, 1, :N, :] + accp[1, 1, :N, :]], axis=1)
    h1 = jnp.maximum(acc * dinv_ref[...] + b1_ref[...], 0.0)
    h2 = jnp.dot(h1, w2_ref[...], preferred_element_type=f32)
    h2_ref[...] = h2
    a2 = jnp.dot(h2, att_ref[...], preferred_element_type=f32)
    pad = jnp.zeros((NPAD - N, 1), f32)
    as_ref[...] = jnp.concatenate([a2[:, 0:1], pad], axis=0)
    ad_ref[...] = jnp.concatenate([a2[:, 1:2], pad], axis=0)


def _tc3_body(nump_ref, denp_ref, b2_ref, out_ref):
    nump = nump_ref[...]
    num = jnp.concatenate(
        [nump[0, 0, :N, :] + nump[1, 0, :N, :],
         nump[0, 1, :N, :] + nump[1, 1, :N, :]], axis=1)
    den = jnp.sum(denp_ref[...][:, :N], axis=0)[:, None]
    out_ref[...] = num / den + b2_ref[...]


_tc1 = pl.pallas_call(
    _tc1_body,
    out_shape=(jax.ShapeDtypeStruct((N, F), f32),
               jax.ShapeDtypeStruct((N, 1), f32)))
_tc2 = pl.pallas_call(
    _tc2_body,
    out_shape=(jax.ShapeDtypeStruct((N, F), f32),
               jax.ShapeDtypeStruct((NPAD, 1), f32),
               jax.ShapeDtypeStruct((NPAD, 1), f32)))
_tc3 = pl.pallas_call(
    _tc3_body,
    out_shape=jax.ShapeDtypeStruct((N, F), f32))


def kernel(x, edge_index, W1, b1, W2, att_src, att_dst, b2):
    loop = jnp.arange(N, dtype=jnp.int32)
    src = jnp.concatenate([edge_index[0], loop])
    dst = jnp.concatenate([edge_index[1], loop])
    padn = TOTCP * CH - ET
    srcp = jnp.concatenate(
        [src, jnp.zeros((padn,), jnp.int32)]).reshape(TOTCP, CH)
    dstp = jnp.concatenate(
        [dst, jnp.full((padn,), N, jnp.int32)]).reshape(TOTCP, CH)
    src2 = jnp.stack([2 * srcp, 2 * srcp + 1])

    zf = jnp.zeros((CH, FH), f32)
    attmat = jnp.stack([att_src, att_dst], axis=1)

    degp = _sc_deg(dstp)
    g, dinv = _tc1(x, W1, degp)
    accp = _sc_gcn(src2, dstp, g.reshape(2 * N, FH), zf)
    h2, asp, adp = _tc2(accp, dinv, b1.reshape(1, F), W2, attmat)
    nump, denp = _sc_gat(src2, dstp, h2.reshape(2 * N, FH),
                         asp.reshape(NPAD), adp.reshape(NPAD), zf)
    out = _tc3(nump, denp, b2.reshape(1, F))
    return out
